# Initial kernel scaffold; baseline (speedup 1.0000x reference)
import functools

import jax
import jax.numpy as jnp
from jax import lax
from jax.experimental import pallas as pl
from jax.experimental.pallas import tpu as pltpu
from jax.experimental.pallas import tpu_sc as plsc

N = 10000
E = 320000
H = 128
NC = 2
NS = 16
NW = NC * NS
EW = E // NW
CHUNK = 80
CHUNKS = EW // CHUNK
RPT = N // NS
NB = 80
NBT = NB // NS


def _mesh():
    return plsc.VectorSubcoreMesh(core_axis_name="c", subcore_axis_name="s")


def _deg_body(dst_hbm, zeros_hbm, out_hbm, idx_v, hist_v, iota_v, shist):
    c = lax.axis_index("c")
    s = lax.axis_index("s")
    w = c * NS + s
    pltpu.sync_copy(dst_hbm.at[pl.ds(w * EW, EW)], idx_v)
    pltpu.sync_copy(zeros_hbm.at[pl.ds(0, NB)], hist_v)
    pltpu.sync_copy(zeros_hbm.at[pl.ds(0, NBT)], shist.at[pl.ds(s * NBT, NBT)])
    for j in range(NB // 16):
        iota_v[pl.ds(j * 16, 16)] = lax.iota(jnp.int32, 16) + j * 16

    ones = jnp.ones((16,), jnp.float32)
    lanes = lax.iota(jnp.int32, 16)

    @pl.loop(0, EW // 16)
    def _(i):
        v = idx_v[pl.ds(i * 16, 16)]
        row = lax.shift_right_logical(v, 7)
        col = lax.bitwise_and(v, 127)
        for j in range(16):
            plsc.addupdate_scatter(hist_v, [row, col], ones, mask=lanes == j)

    plsc.subcore_barrier()
    pltpu.sync_copy(hist_v, shist.at[iota_v], add=True)
    plsc.subcore_barrier()
    pltpu.sync_copy(shist.at[pl.ds(s * NBT, NBT)],
                    out_hbm.at[c, pl.ds(s * NBT, NBT)])


def _sc_degree(dst, zeros):
    k = pl.kernel(
        _deg_body,
        out_type=jax.ShapeDtypeStruct((NC, NB, 128), jnp.float32),
        mesh=_mesh(),
        scratch_types=[
            pltpu.VMEM((EW,), jnp.int32),
            pltpu.VMEM((NB, 128), jnp.float32),
            pltpu.VMEM((NB,), jnp.int32),
            pltpu.VMEM_SHARED((NB, 128), jnp.float32),
        ],
    )
    return k(dst, zeros)


def _agg_body(ys_hbm, src_hbm, dst3_hbm, zeros_hbm, out_hbm,
              isrc_v, idst_v, rows_v, acc):
    c = lax.axis_index("c")
    s = lax.axis_index("s")
    w = c * NS + s
    pltpu.sync_copy(zeros_hbm.at[pl.ds(s * RPT, RPT)],
                    acc.at[pl.ds(s * RPT, RPT)])
    pltpu.sync_copy(src_hbm.at[pl.ds(w * EW, EW)], isrc_v)
    pltpu.sync_copy(dst3_hbm.at[w], idst_v)
    plsc.subcore_barrier()

    @pl.loop(0, CHUNKS)
    def _(k):
        pltpu.sync_copy(ys_hbm.at[isrc_v.at[pl.ds(k * CHUNK, CHUNK)]], rows_v)
        pltpu.sync_copy(rows_v, acc.at[idst_v.at[k]], add=True)

    plsc.subcore_barrier()
    pltpu.sync_copy(acc.at[pl.ds(s * RPT, RPT)],
                    out_hbm.at[c, pl.ds(s * RPT, RPT)])


def _sc_agg(ys, src, dst3, zeros):
    k = pl.kernel(
        _agg_body,
        out_type=jax.ShapeDtypeStruct((NC, N, H), jnp.float32),
        mesh=_mesh(),
        scratch_types=[
            pltpu.VMEM((EW,), jnp.int32),
            pltpu.VMEM((CHUNKS, CHUNK), jnp.int32),
            pltpu.VMEM((CHUNK, H), jnp.float32),
            pltpu.VMEM_SHARED((N, H), jnp.float32),
        ],
    )
    return k(ys, src, dst3, zeros)


def _elu(x):
    return jnp.where(x > 0, x, jnp.exp(jnp.minimum(x, 0.0)) - 1.0)


def _tc_in_body(ctrl_ref, win_ref, bin_ref, lng_ref, lnb_ref, w1_ref, deg_ref,
                ys1_ref, dinv_ref):
    x = ctrl_ref[...] * win_ref[...] + bin_ref[...]
    m = jnp.mean(x, axis=-1, keepdims=True)
    v = jnp.mean((x - m) ** 2, axis=-1, keepdims=True)
    x = (x - m) * lax.rsqrt(v + 1e-5) * lng_ref[...] + lnb_ref[...]
    x = _elu(x)
    dinv = lax.rsqrt(deg_ref[0] + deg_ref[1] + 1.0)
    dinv_ref[...] = dinv
    ys1_ref[...] = jnp.dot(x, w1_ref[...],
                           preferred_element_type=jnp.float32) * dinv


def _tc_input(ctrl_col, W_in, b_in2, ln_g2, ln_b2, W1, degcol):
    return pl.pallas_call(
        _tc_in_body,
        out_shape=[jax.ShapeDtypeStruct((N, H), jnp.float32),
                   jax.ShapeDtypeStruct((N, 1), jnp.float32)],
    )(ctrl_col, W_in, b_in2, ln_g2, ln_b2, W1, degcol)


def _tc_mid_body(parts_ref, ys_ref, dinv_ref, b_ref, g_ref, be_ref, w_ref,
                 ysn_ref):
    dinv = dinv_ref[...]
    z = (parts_ref[0] + parts_ref[1] + ys_ref[...]) * dinv + b_ref[...]
    m = jnp.mean(z, axis=0, keepdims=True)
    v = jnp.mean((z - m) ** 2, axis=0, keepdims=True)
    x = _elu((z - m) * lax.rsqrt(v + 1e-5) * g_ref[...] + be_ref[...])
    ysn_ref[...] = jnp.dot(x, w_ref[...],
                           preferred_element_type=jnp.float32) * dinv


def _tc_mid(parts, ys, dinv, b2d, g2d, be2d, Wn):
    return pl.pallas_call(
        _tc_mid_body,
        out_shape=jax.ShapeDtypeStruct((N, H), jnp.float32),
    )(parts, ys, dinv, b2d, g2d, be2d, Wn)


def _tc_out_body(parts_ref, ys_ref, dinv_ref, b_ref, g_ref, be_ref, wh_ref,
                 bh_ref, out_ref):
    z = (parts_ref[0] + parts_ref[1] + ys_ref[...]) * dinv_ref[...] + b_ref[...]
    m = jnp.mean(z, axis=0, keepdims=True)
    v = jnp.mean((z - m) ** 2, axis=0, keepdims=True)
    x = _elu((z - m) * lax.rsqrt(v + 1e-5) * g_ref[...] + be_ref[...])
    out_ref[...] = jnp.dot(x, wh_ref[...],
                           preferred_element_type=jnp.float32) + bh_ref[...]


def _tc_out(parts, ys, dinv, b2d, g2d, be2d, W_head, bh2d):
    return pl.pallas_call(
        _tc_out_body,
        out_shape=jax.ShapeDtypeStruct((N, 1), jnp.float32),
    )(parts, ys, dinv, b2d, g2d, be2d, W_head, bh2d)


def kernel(ctrl_expr, W_in, b_in, ln_g, ln_b, W1, b1, g1, be1, W2, b2, g2,
           be2, W3, b3, g3, be3, W_head, b_head, edge_index):
    src = edge_index[0]
    dst = edge_index[1]
    dst3 = dst.reshape(NW, CHUNKS, CHUNK)
    zeros = jnp.zeros((N, H), jnp.float32)

    degp = _sc_degree(dst, zeros)
    degcol = degp.reshape(NC, NB * 128)[:, :N, None]

    row = lambda a: a.reshape(1, -1)
    ys1, dinv = _tc_input(ctrl_expr.reshape(N, 1), W_in, row(b_in),
                          row(ln_g), row(ln_b), W1, degcol)
    p1 = _sc_agg(ys1, src, dst3, zeros)
    ys2 = _tc_mid(p1, ys1, dinv, row(b1), row(g1), row(be1), W2)
    p2 = _sc_agg(ys2, src, dst3, zeros)
    ys3 = _tc_mid(p2, ys2, dinv, row(b2), row(g2), row(be2), W3)
    p3 = _sc_agg(ys3, src, dst3, zeros)
    out = _tc_out(p3, ys3, dinv, row(b3), row(g3), row(be3), W_head,
                  b_head.reshape(1, 1))
    return out.reshape(N)


# speedup vs baseline: 14.0254x; 14.0254x over previous
import functools

import jax
import jax.numpy as jnp
from jax import lax
from jax.experimental import pallas as pl
from jax.experimental.pallas import tpu as pltpu
from jax.experimental.pallas import tpu_sc as plsc

N = 10000
NP = 10240
E = 320000
H = 128
NC = 2
NS = 16
NW = NC * NS
EW = E // NW
CHUNK = 80
CHUNKS = EW // CHUNK
RPT = NP // NS
NB = 128
NBT = NB // NS


def _mesh():
    return plsc.VectorSubcoreMesh(core_axis_name="c", subcore_axis_name="s")


DW = 128


def _deg_body(dst3_hbm, ones_hbm, zeros_hbm, out_hbm, idst_v, ones_v, acc):
    c = lax.axis_index("c")
    s = lax.axis_index("s")
    w = c * NS + s
    pltpu.sync_copy(zeros_hbm.at[pl.ds(s * RPT, RPT)],
                    acc.at[pl.ds(s * RPT, RPT)])
    pltpu.sync_copy(ones_hbm, ones_v)
    pltpu.sync_copy(dst3_hbm.at[w], idst_v)
    plsc.subcore_barrier()

    @pl.loop(0, CHUNKS)
    def _(k):
        pltpu.sync_copy(ones_v, acc.at[idst_v.at[k]], add=True)

    plsc.subcore_barrier()
    pltpu.sync_copy(acc.at[pl.ds(s * RPT, RPT)],
                    out_hbm.at[c, pl.ds(s * RPT, RPT)])


def _sc_degree(dst3, ones16, zeros16):
    k = pl.kernel(
        _deg_body,
        out_type=jax.ShapeDtypeStruct((NC, NP, DW), jnp.float32),
        mesh=_mesh(),
        scratch_types=[
            pltpu.VMEM((CHUNKS, CHUNK), jnp.int32),
            pltpu.VMEM((CHUNK, DW), jnp.float32),
            pltpu.VMEM_SHARED((NP, DW), jnp.float32),
        ],
    )
    return k(dst3, ones16, zeros16)


def _agg_body(ys_hbm, src_hbm, dst3_hbm, zeros_hbm, out_hbm,
              isrc_v, idst_v, rows_v, acc):
    c = lax.axis_index("c")
    s = lax.axis_index("s")
    w = c * NS + s
    pltpu.sync_copy(zeros_hbm.at[pl.ds(s * RPT, RPT)],
                    acc.at[pl.ds(s * RPT, RPT)])
    pltpu.sync_copy(src_hbm.at[pl.ds(w * EW, EW)], isrc_v)
    pltpu.sync_copy(dst3_hbm.at[w], idst_v)
    plsc.subcore_barrier()

    @pl.loop(0, CHUNKS)
    def _(k):
        pltpu.sync_copy(ys_hbm.at[isrc_v.at[pl.ds(k * CHUNK, CHUNK)]], rows_v)
        pltpu.sync_copy(rows_v, acc.at[idst_v.at[k]], add=True)

    plsc.subcore_barrier()
    pltpu.sync_copy(acc.at[pl.ds(s * RPT, RPT)],
                    out_hbm.at[c, pl.ds(s * RPT, RPT)])


def _sc_agg(ys, src, dst3, zeros):
    k = pl.kernel(
        _agg_body,
        out_type=jax.ShapeDtypeStruct((NC, NP, H), jnp.float32),
        mesh=_mesh(),
        scratch_types=[
            pltpu.VMEM((EW,), jnp.int32),
            pltpu.VMEM((CHUNKS, CHUNK), jnp.int32),
            pltpu.VMEM((CHUNK, H), jnp.float32),
            pltpu.VMEM_SHARED((NP, H), jnp.float32),
        ],
    )
    return k(ys, src, dst3, zeros)


def _elu(x):
    return jnp.where(x > 0, x, jnp.exp(jnp.minimum(x, 0.0)) - 1.0)


def _tc_in_body(ctrl_ref, win_ref, bin_ref, lng_ref, lnb_ref, w1_ref, deg_ref,
                ys1_ref, dinv_ref):
    x = ctrl_ref[...] * win_ref[...] + bin_ref[...]
    m = jnp.mean(x, axis=-1, keepdims=True)
    v = jnp.mean((x - m) ** 2, axis=-1, keepdims=True)
    x = (x - m) * lax.rsqrt(v + 1e-5) * lng_ref[...] + lnb_ref[...]
    x = _elu(x)
    dinv = lax.rsqrt(deg_ref[0] + deg_ref[1] + 1.0)
    dinv_ref[...] = dinv
    ys1_ref[...] = jnp.dot(x, w1_ref[...],
                           preferred_element_type=jnp.float32) * dinv


def _tc_input(ctrl_col, W_in, b_in2, ln_g2, ln_b2, W1, degcol):
    return pl.pallas_call(
        _tc_in_body,
        out_shape=[jax.ShapeDtypeStruct((N, H), jnp.float32),
                   jax.ShapeDtypeStruct((N, 1), jnp.float32)],
    )(ctrl_col, W_in, b_in2, ln_g2, ln_b2, W1, degcol)


def _tc_mid_body(parts_ref, ys_ref, dinv_ref, b_ref, g_ref, be_ref, w_ref,
                 ysn_ref):
    dinv = dinv_ref[...]
    z = (parts_ref[0] + parts_ref[1] + ys_ref[...]) * dinv + b_ref[...]
    m = jnp.mean(z, axis=0, keepdims=True)
    v = jnp.mean((z - m) ** 2, axis=0, keepdims=True)
    x = _elu((z - m) * lax.rsqrt(v + 1e-5) * g_ref[...] + be_ref[...])
    ysn_ref[...] = jnp.dot(x, w_ref[...],
                           preferred_element_type=jnp.float32) * dinv


def _tc_mid(parts, ys, dinv, b2d, g2d, be2d, Wn):
    return pl.pallas_call(
        _tc_mid_body,
        out_shape=jax.ShapeDtypeStruct((N, H), jnp.float32),
    )(parts, ys, dinv, b2d, g2d, be2d, Wn)


def _tc_out_body(parts_ref, ys_ref, dinv_ref, b_ref, g_ref, be_ref, wh_ref,
                 bh_ref, out_ref):
    z = (parts_ref[0] + parts_ref[1] + ys_ref[...]) * dinv_ref[...] + b_ref[...]
    m = jnp.mean(z, axis=0, keepdims=True)
    v = jnp.mean((z - m) ** 2, axis=0, keepdims=True)
    x = _elu((z - m) * lax.rsqrt(v + 1e-5) * g_ref[...] + be_ref[...])
    out_ref[...] = jnp.dot(x, wh_ref[...],
                           preferred_element_type=jnp.float32) + bh_ref[...]


def _tc_out(parts, ys, dinv, b2d, g2d, be2d, W_head, bh2d):
    return pl.pallas_call(
        _tc_out_body,
        out_shape=jax.ShapeDtypeStruct((N, 1), jnp.float32),
    )(parts, ys, dinv, b2d, g2d, be2d, W_head, bh2d)


def kernel(ctrl_expr, W_in, b_in, ln_g, ln_b, W1, b1, g1, be1, W2, b2, g2,
           be2, W3, b3, g3, be3, W_head, b_head, edge_index):
    src = edge_index[0]
    dst = edge_index[1]
    dst3 = dst.reshape(NW, CHUNKS, CHUNK)
    zeros = jnp.zeros((NP, H), jnp.float32)

    degp = _sc_degree(dst3, jnp.ones((CHUNK, DW), jnp.float32),
                      jnp.zeros((NP, DW), jnp.float32))
    degcol = degp[:, :N, 0:1]

    row = lambda a: a.reshape(1, -1)
    ys1, dinv = _tc_input(ctrl_expr.reshape(N, 1), W_in, row(b_in),
                          row(ln_g), row(ln_b), W1, degcol)
    p1 = _sc_agg(ys1, src, dst3, zeros)[:, :N]
    ys2 = _tc_mid(p1, ys1, dinv, row(b1), row(g1), row(be1), W2)
    p2 = _sc_agg(ys2, src, dst3, zeros)[:, :N]
    ys3 = _tc_mid(p2, ys2, dinv, row(b2), row(g2), row(be2), W3)
    p3 = _sc_agg(ys3, src, dst3, zeros)[:, :N]
    out = _tc_out(p3, ys3, dinv, row(b3), row(g3), row(be3), W_head,
                  b_head.reshape(1, 1))
    return out.reshape(N)
